# Initial kernel scaffold; baseline (speedup 1.0000x reference)
import jax
import jax.numpy as jnp
from jax.experimental import pallas as pl


def kernel(predicted_coords, actual_coords, coord_mask):
    raise NotImplementedError("write your pallas kernel here")


# speedup vs baseline: 1.7229x; 1.7229x over previous
import functools

import jax
import jax.numpy as jnp
from jax.experimental import pallas as pl
from jax.experimental.pallas import tpu as pltpu


def _tile_kernel(inv_d0, pcols_ref, ncols_ref, prows_ref, nrows_ref,
                 sum_ref, cnt_ref):
    pc = pcols_ref[0]
    nc = ncols_ref[0]
    pr = prows_ref[0]
    nr = nrows_ref[0]

    mcol = pc[3:4, :]
    mrow = pr[:, 3:4]

    dsq_p = (pr[:, 0:1] - pc[0:1, :]) ** 2
    dsq_p += (pr[:, 1:2] - pc[1:2, :]) ** 2
    dsq_p += (pr[:, 2:3] - pc[2:3, :]) ** 2

    dsq_n = (nr[:, 0:1] - nc[0:1, :]) ** 2
    dsq_n += (nr[:, 1:2] - nc[1:2, :]) ** 2
    dsq_n += (nr[:, 2:3] - nc[2:3, :]) ** 2

    delta = (jnp.sqrt(dsq_p) - jnp.sqrt(dsq_n)) * inv_d0
    prox = 1.0 / (1.0 + delta * delta)

    psum = jnp.sum((prox * mcol) * mrow)
    csum = jnp.sum(mrow) * jnp.sum(mcol)

    sum_ref[...] = jnp.full((1, 1, 128), psum, jnp.float32)
    cnt_ref[...] = jnp.full((1, 1, 128), csum, jnp.float32)


def kernel(predicted_coords, actual_coords, coord_mask):
    b, n_res, n_atoms, _ = predicted_coords.shape
    n = n_res * n_atoms
    d0 = 1.24 * (n_res - 15.0) ** (1.0 / 3.0) - 1.8
    inv_d0 = float(1.0 / d0)

    block_rows = 256
    rb = n // block_rows
    grid = (b, rb)
    g = b * rb

    pred3 = predicted_coords.reshape(b, n, 3).astype(jnp.float32)
    nat3 = actual_coords.reshape(b, n, 3).astype(jnp.float32)
    maskf = coord_mask.reshape(b, n).astype(jnp.float32)

    zc = jnp.zeros((b, 4, n), jnp.float32)
    pcols = jnp.concatenate(
        [pred3.transpose(0, 2, 1), maskf[:, None, :], zc], axis=1)
    ncols = jnp.concatenate(
        [nat3.transpose(0, 2, 1), maskf[:, None, :], zc], axis=1)
    zr = jnp.zeros((b, n, 124), jnp.float32)
    prows = jnp.concatenate([pred3, maskf[:, :, None], zr], axis=2)
    nrows = jnp.concatenate([nat3, maskf[:, :, None], zr], axis=2)

    psums, csums = pl.pallas_call(
        functools.partial(_tile_kernel, inv_d0),
        grid=grid,
        in_specs=[
            pl.BlockSpec((1, 8, n), lambda i, j: (i, 0, 0)),
            pl.BlockSpec((1, 8, n), lambda i, j: (i, 0, 0)),
            pl.BlockSpec((1, block_rows, 128), lambda i, j: (i, j, 0)),
            pl.BlockSpec((1, block_rows, 128), lambda i, j: (i, j, 0)),
        ],
        out_specs=[
            pl.BlockSpec((1, 1, 128), lambda i, j, rb=rb: (i * rb + j, 0, 0)),
            pl.BlockSpec((1, 1, 128), lambda i, j, rb=rb: (i * rb + j, 0, 0)),
        ],
        out_shape=[
            jax.ShapeDtypeStruct((g, 1, 128), jnp.float32),
            jax.ShapeDtypeStruct((g, 1, 128), jnp.float32),
        ],
        compiler_params=pltpu.CompilerParams(
            dimension_semantics=("parallel", "arbitrary"),
        ),
        name="distance_inv_loss",
    )(pcols, ncols, prows, nrows)

    total = jnp.sum(psums[:, 0, 0])
    count = jnp.sum(csums[:, 0, 0])
    return -(total / count)


# speedup vs baseline: 1.9849x; 1.1521x over previous
import functools

import jax
import jax.numpy as jnp
from jax.experimental import pallas as pl
from jax.experimental.pallas import tpu as pltpu


def _tile_kernel(inv_d0, pcols_ref, ncols_ref, prows_ref, nrows_ref,
                 sum_ref, cnt_ref):
    pc = pcols_ref[0]
    nc = ncols_ref[0]
    pr = prows_ref[0]
    nr = nrows_ref[0]

    mcol = pc[3:4, :]
    mrow = pr[:, 3:4]

    dsq_p = (pr[:, 0:1] - pc[0:1, :]) ** 2
    dsq_p += (pr[:, 1:2] - pc[1:2, :]) ** 2
    dsq_p += (pr[:, 2:3] - pc[2:3, :]) ** 2

    dsq_n = (nr[:, 0:1] - nc[0:1, :]) ** 2
    dsq_n += (nr[:, 1:2] - nc[1:2, :]) ** 2
    dsq_n += (nr[:, 2:3] - nc[2:3, :]) ** 2

    dp = dsq_p * jax.lax.rsqrt(dsq_p + 1e-12)
    dn = dsq_n * jax.lax.rsqrt(dsq_n + 1e-12)
    delta = (dp - dn) * inv_d0
    prox = 1.0 / (1.0 + delta * delta)

    psum = jnp.sum((prox * mcol) * mrow)
    csum = jnp.sum(mrow) * jnp.sum(mcol)

    sum_ref[...] = jnp.full((1, 1, 128), psum, jnp.float32)
    cnt_ref[...] = jnp.full((1, 1, 128), csum, jnp.float32)


def kernel(predicted_coords, actual_coords, coord_mask):
    b, n_res, n_atoms, _ = predicted_coords.shape
    n = n_res * n_atoms
    d0 = 1.24 * (n_res - 15.0) ** (1.0 / 3.0) - 1.8
    inv_d0 = float(1.0 / d0)

    block_rows = 256
    rb = n // block_rows
    grid = (b, rb)
    g = b * rb

    pred3 = predicted_coords.reshape(b, n, 3).astype(jnp.float32)
    nat3 = actual_coords.reshape(b, n, 3).astype(jnp.float32)
    maskf = coord_mask.reshape(b, n).astype(jnp.float32)

    zc = jnp.zeros((b, 4, n), jnp.float32)
    pcols = jnp.concatenate(
        [pred3.transpose(0, 2, 1), maskf[:, None, :], zc], axis=1)
    ncols = jnp.concatenate(
        [nat3.transpose(0, 2, 1), maskf[:, None, :], zc], axis=1)
    zr = jnp.zeros((b, n, 124), jnp.float32)
    prows = jnp.concatenate([pred3, maskf[:, :, None], zr], axis=2)
    nrows = jnp.concatenate([nat3, maskf[:, :, None], zr], axis=2)

    psums, csums = pl.pallas_call(
        functools.partial(_tile_kernel, inv_d0),
        grid=grid,
        in_specs=[
            pl.BlockSpec((1, 8, n), lambda i, j: (i, 0, 0)),
            pl.BlockSpec((1, 8, n), lambda i, j: (i, 0, 0)),
            pl.BlockSpec((1, block_rows, 128), lambda i, j: (i, j, 0)),
            pl.BlockSpec((1, block_rows, 128), lambda i, j: (i, j, 0)),
        ],
        out_specs=[
            pl.BlockSpec((1, 1, 128), lambda i, j, rb=rb: (i * rb + j, 0, 0)),
            pl.BlockSpec((1, 1, 128), lambda i, j, rb=rb: (i * rb + j, 0, 0)),
        ],
        out_shape=[
            jax.ShapeDtypeStruct((g, 1, 128), jnp.float32),
            jax.ShapeDtypeStruct((g, 1, 128), jnp.float32),
        ],
        compiler_params=pltpu.CompilerParams(
            dimension_semantics=("parallel", "arbitrary"),
        ),
        name="distance_inv_loss",
    )(pcols, ncols, prows, nrows)

    total = jnp.sum(psums[:, 0, 0])
    count = jnp.sum(csums[:, 0, 0])
    return -(total / count)


# speedup vs baseline: 2.1547x; 1.0855x over previous
import functools

import jax
import jax.numpy as jnp
from jax.experimental import pallas as pl
from jax.experimental.pallas import tpu as pltpu


def _tile_kernel(inv_d0, block_rows, a_ref, sum_ref, cnt_ref):
    j = pl.program_id(1)
    a = a_ref[0]
    rs = a_ref[0, :, pl.ds(j * block_rows, block_rows)]
    rt = jnp.transpose(rs)

    mcol = a[3:4, :]
    mrow = rt[:, 3:4]

    dsq_p = (rt[:, 0:1] - a[0:1, :]) ** 2
    dsq_p += (rt[:, 1:2] - a[1:2, :]) ** 2
    dsq_p += (rt[:, 2:3] - a[2:3, :]) ** 2

    dsq_n = (rt[:, 4:5] - a[4:5, :]) ** 2
    dsq_n += (rt[:, 5:6] - a[5:6, :]) ** 2
    dsq_n += (rt[:, 6:7] - a[6:7, :]) ** 2

    dp = dsq_p * jax.lax.rsqrt(dsq_p + 1e-12)
    dn = dsq_n * jax.lax.rsqrt(dsq_n + 1e-12)
    delta = (dp - dn) * inv_d0
    prox = 1.0 / (1.0 + delta * delta)

    psum = jnp.sum((prox * mcol) * mrow)
    csum = jnp.sum(mrow) * jnp.sum(mcol)

    sum_ref[...] = jnp.full((1, 1, 128), psum, jnp.float32)
    cnt_ref[...] = jnp.full((1, 1, 128), csum, jnp.float32)


def kernel(predicted_coords, actual_coords, coord_mask):
    b, n_res, n_atoms, _ = predicted_coords.shape
    n = n_res * n_atoms
    d0 = 1.24 * (n_res - 15.0) ** (1.0 / 3.0) - 1.8
    inv_d0 = float(1.0 / d0)

    block_rows = 256
    rb = n // block_rows
    grid = (b, rb)
    g = b * rb

    pred3 = predicted_coords.reshape(b, n, 3).astype(jnp.float32)
    nat3 = actual_coords.reshape(b, n, 3).astype(jnp.float32)
    maskf = coord_mask.reshape(b, 1, n).astype(jnp.float32)

    packed = jnp.concatenate(
        [pred3.transpose(0, 2, 1), maskf,
         nat3.transpose(0, 2, 1), jnp.zeros((b, 1, n), jnp.float32)],
        axis=1)

    psums, csums = pl.pallas_call(
        functools.partial(_tile_kernel, inv_d0, block_rows),
        grid=grid,
        in_specs=[
            pl.BlockSpec((1, 8, n), lambda i, j: (i, 0, 0)),
        ],
        out_specs=[
            pl.BlockSpec((1, 1, 128), lambda i, j, rb=rb: (i * rb + j, 0, 0)),
            pl.BlockSpec((1, 1, 128), lambda i, j, rb=rb: (i * rb + j, 0, 0)),
        ],
        out_shape=[
            jax.ShapeDtypeStruct((g, 1, 128), jnp.float32),
            jax.ShapeDtypeStruct((g, 1, 128), jnp.float32),
        ],
        compiler_params=pltpu.CompilerParams(
            dimension_semantics=("parallel", "arbitrary"),
        ),
        name="distance_inv_loss",
    )(packed)

    total = jnp.sum(psums[:, 0, 0])
    count = jnp.sum(csums[:, 0, 0])
    return -(total / count)
